# baseline (speedup 1.0000x reference)
import functools

import jax
import jax.numpy as jnp
from jax import lax
from jax.experimental import pallas as pl
from jax.experimental.pallas import tpu as pltpu
from jax.experimental.pallas import tpu_sc as plsc

_BLK = 2048
_IDX_CHUNK = 128


@functools.lru_cache(maxsize=None)
def _make_sc_gather(V, D, B):
    info = plsc.get_sparse_core_info()
    NC, NS = info.num_cores, info.num_subcores
    NW = NC * NS
    bw = B // NW
    nch = bw // _IDX_CHUNK
    mesh = plsc.VectorSubcoreMesh(core_axis_name="c", subcore_axis_name="s")

    @functools.partial(
        pl.kernel,
        mesh=mesh,
        out_type=jax.ShapeDtypeStruct((B, D), jnp.float32),
        scratch_types=[
            pltpu.VMEM((bw,), jnp.int32),
            pltpu.VMEM((bw, D), jnp.float32),
            pltpu.SemaphoreType.DMA((8,)),
            pltpu.SemaphoreType.DMA,
        ],
    )
    def gather_k(table_hbm, idx_hbm, out_hbm, idx_v, rows_v, gsem, wsem):
        wid = lax.axis_index("s") * NC + lax.axis_index("c")
        base = wid * bw
        pltpu.sync_copy(idx_hbm.at[pl.ds(base, bw)], idx_v)
        gathers = [
            pltpu.async_copy(
                table_hbm.at[idx_v.at[pl.ds(j * _IDX_CHUNK, _IDX_CHUNK)]],
                rows_v.at[pl.ds(j * _IDX_CHUNK, _IDX_CHUNK)],
                gsem.at[j],
            )
            for j in range(nch)
        ]
        writes = []
        for j in range(nch):
            gathers[j].wait()
            writes.append(pltpu.async_copy(
                rows_v.at[pl.ds(j * _IDX_CHUNK, _IDX_CHUNK)],
                out_hbm.at[pl.ds(base + j * _IDX_CHUNK, _IDX_CHUNK)],
                wsem,
            ))
        for w in writes:
            w.wait()

    return gather_k


def _dense_body(maskf_ref, clsf_ref, desc_ref, chem_ref, ctab_ref, w1_ref,
                b1_ref, w2_ref, wgcb_ref, wgfb_ref, jm_ref, bg_ref,
                gamma_ref, beta_ref, out_ref):
    blk = chem_ref.shape[0]
    chem = chem_ref[...]
    desc = desc_ref[...]

    mask_col = maskf_ref[...].reshape(blk, 1)
    cls_col = clsf_ref[...].reshape(blk, 1)
    ones_row = jnp.ones((1, ctab_ref.shape[0]), jnp.float32)
    cls_dense = jnp.dot(cls_col, ones_row, preferred_element_type=jnp.float32)
    onehot = (cls_dense == lax.broadcasted_iota(
        jnp.int32, cls_dense.shape, 1).astype(jnp.float32)).astype(jnp.float32)
    cls_emb = jnp.dot(onehot, ctab_ref[...], preferred_element_type=jnp.float32)

    h = jnp.maximum(
        jnp.dot(desc, w1_ref[...], preferred_element_type=jnp.float32) + b1_ref[...],
        0.0)
    fallback = (jnp.dot(h, w2_ref[...], preferred_element_type=jnp.float32)
                + cls_emb)

    logit = (jnp.dot(chem, wgcb_ref[...], preferred_element_type=jnp.float32)
             + jnp.dot(fallback, wgfb_ref[...], preferred_element_type=jnp.float32)
             + bg_ref[0, 0])
    mask_dense = jnp.dot(mask_col, jnp.ones((1, chem.shape[1]), jnp.float32),
                         preferred_element_type=jnp.float32)
    alpha = mask_dense / (1.0 + jnp.exp(-logit))

    emb = fallback + alpha * (chem - fallback)
    mean = jnp.dot(emb, jm_ref[...], preferred_element_type=jnp.float32)
    cent = emb - mean
    var = jnp.dot(cent * cent, jm_ref[...], preferred_element_type=jnp.float32)
    out_ref[...] = cent * lax.rsqrt(var + 1e-5) * gamma_ref[...] + beta_ref[...]


def _dense_forward(maskf, clsf, desc, chem_emb, ctab, w1, b1, w2,
                   wgcb, wgfb, jm, bg2, gamma2, beta2):
    B, D = chem_emb.shape
    nd = desc.shape[1]
    nb = B // _BLK

    def row_blk(i):
        return (i, 0)

    def row_blk1(i):
        return (i,)

    def fixed(i):
        return (0, 0)

    in_specs = [
        pl.BlockSpec((_BLK,), row_blk1),
        pl.BlockSpec((_BLK,), row_blk1),
        pl.BlockSpec((_BLK, nd), row_blk),
        pl.BlockSpec((_BLK, D), row_blk),
        pl.BlockSpec(ctab.shape, fixed),
        pl.BlockSpec(w1.shape, fixed),
        pl.BlockSpec(b1.shape, fixed),
        pl.BlockSpec(w2.shape, fixed),
        pl.BlockSpec(wgcb.shape, fixed),
        pl.BlockSpec(wgfb.shape, fixed),
        pl.BlockSpec(jm.shape, fixed),
        pl.BlockSpec(bg2.shape, fixed),
        pl.BlockSpec(gamma2.shape, fixed),
        pl.BlockSpec(beta2.shape, fixed),
    ]
    out_shape = jax.ShapeDtypeStruct((B, D), jnp.float32)
    out_specs = pl.BlockSpec((_BLK, D), row_blk)
    return pl.pallas_call(
        _dense_body,
        grid=(nb,),
        in_specs=in_specs,
        out_specs=out_specs,
        out_shape=out_shape,
    )(maskf, clsf, desc, chem_emb, ctab, w1, b1, w2, wgcb, wgfb, jm, bg2,
      gamma2, beta2)


def kernel(chemical_idx, class_idx, descriptors, chem_table, class_table,
           W1, b1, W2, b2, Wg, bg, gamma, beta):
    B = chemical_idx.shape[0]
    V, D = chem_table.shape

    idx32 = chemical_idx.astype(jnp.int32)
    chem_emb = _make_sc_gather(V, D, B)(chem_table, idx32)

    is_unknown = chemical_idx == 0
    maskf = 1.0 - is_unknown.astype(jnp.float32)
    clsf = class_idx.astype(jnp.float32)
    nclass = class_table.shape[0]
    pad = (-nclass) % 32
    ctab = jnp.pad(class_table + b2.reshape(1, D), ((0, pad), (0, 0)))
    ones_row = jnp.ones((1, D), jnp.float32)
    wgcb = Wg[:D].reshape(D, 1) @ ones_row
    wgfb = Wg[D:].reshape(D, 1) @ ones_row
    jm = jnp.full((D, D), 1.0 / D, jnp.float32)
    out = _dense_forward(
        maskf, clsf, descriptors, chem_emb, ctab, W1, b1.reshape(1, -1), W2,
        wgcb, wgfb, jm, bg.reshape(1, 1), gamma.reshape(1, D),
        beta.reshape(1, D))
    return out, is_unknown


# speedup vs baseline: 1.0327x; 1.0327x over previous
import functools

import jax
import jax.numpy as jnp
from jax import lax
from jax.experimental import pallas as pl
from jax.experimental.pallas import tpu as pltpu
from jax.experimental.pallas import tpu_sc as plsc

_BLK = 4096
_IDX_CHUNK = 128


@functools.lru_cache(maxsize=None)
def _make_sc_gather(V, D, B):
    info = plsc.get_sparse_core_info()
    NC, NS = info.num_cores, info.num_subcores
    NW = NC * NS
    bw = B // NW
    nch = bw // _IDX_CHUNK
    mesh = plsc.VectorSubcoreMesh(core_axis_name="c", subcore_axis_name="s")

    @functools.partial(
        pl.kernel,
        mesh=mesh,
        out_type=jax.ShapeDtypeStruct((B, D), jnp.float32),
        scratch_types=[
            pltpu.VMEM((bw,), jnp.int32),
            pltpu.VMEM((bw, D), jnp.float32),
            pltpu.SemaphoreType.DMA((8,)),
            pltpu.SemaphoreType.DMA,
        ],
    )
    def gather_k(table_hbm, idx_hbm, out_hbm, idx_v, rows_v, gsem, wsem):
        wid = lax.axis_index("s") * NC + lax.axis_index("c")
        base = wid * bw
        pltpu.sync_copy(idx_hbm.at[pl.ds(base, bw)], idx_v)
        gathers = [
            pltpu.async_copy(
                table_hbm.at[idx_v.at[pl.ds(j * _IDX_CHUNK, _IDX_CHUNK)]],
                rows_v.at[pl.ds(j * _IDX_CHUNK, _IDX_CHUNK)],
                gsem.at[j],
            )
            for j in range(nch)
        ]
        writes = []
        for j in range(nch):
            gathers[j].wait()
            writes.append(pltpu.async_copy(
                rows_v.at[pl.ds(j * _IDX_CHUNK, _IDX_CHUNK)],
                out_hbm.at[pl.ds(base + j * _IDX_CHUNK, _IDX_CHUNK)],
                wsem,
            ))
        for w in writes:
            w.wait()

    return gather_k


def _dense_body(maskf_ref, clsf_ref, desc_ref, chem_ref, ctab_ref, w1_ref,
                b1_ref, w2_ref, wgcb_ref, wgfb_ref, jm_ref, bg_ref,
                gamma_ref, beta_ref, out_ref):
    blk = chem_ref.shape[0]
    chem = chem_ref[...]
    desc = desc_ref[...]

    mask_col = maskf_ref[...].reshape(blk, 1)
    cls_col = clsf_ref[...].reshape(blk, 1)
    ones_row = jnp.ones((1, ctab_ref.shape[0]), jnp.float32)
    cls_dense = jnp.dot(cls_col, ones_row, preferred_element_type=jnp.float32)
    onehot = (cls_dense == lax.broadcasted_iota(
        jnp.int32, cls_dense.shape, 1).astype(jnp.float32)).astype(jnp.float32)
    cls_emb = jnp.dot(onehot, ctab_ref[...], preferred_element_type=jnp.float32)

    h = jnp.maximum(
        jnp.dot(desc, w1_ref[...], preferred_element_type=jnp.float32) + b1_ref[...],
        0.0)
    fallback = (jnp.dot(h, w2_ref[...], preferred_element_type=jnp.float32)
                + cls_emb)

    logit = (jnp.dot(chem, wgcb_ref[...], preferred_element_type=jnp.float32)
             + jnp.dot(fallback, wgfb_ref[...], preferred_element_type=jnp.float32)
             + bg_ref[0, 0])
    mask_dense = jnp.dot(mask_col, jnp.ones((1, chem.shape[1]), jnp.float32),
                         preferred_element_type=jnp.float32)
    alpha = mask_dense / (1.0 + jnp.exp(-logit))

    emb = fallback + alpha * (chem - fallback)
    mean = jnp.dot(emb, jm_ref[...], preferred_element_type=jnp.float32)
    cent = emb - mean
    var = jnp.dot(cent * cent, jm_ref[...], preferred_element_type=jnp.float32)
    out_ref[...] = cent * lax.rsqrt(var + 1e-5) * gamma_ref[...] + beta_ref[...]


def _dense_forward(maskf, clsf, desc, chem_emb, ctab, w1, b1, w2,
                   wgcb, wgfb, jm, bg2, gamma2, beta2):
    B, D = chem_emb.shape
    nd = desc.shape[1]
    nb = B // _BLK

    def row_blk(i):
        return (i, 0)

    def row_blk1(i):
        return (i,)

    def fixed(i):
        return (0, 0)

    in_specs = [
        pl.BlockSpec((_BLK,), row_blk1),
        pl.BlockSpec((_BLK,), row_blk1),
        pl.BlockSpec((_BLK, nd), row_blk),
        pl.BlockSpec((_BLK, D), row_blk),
        pl.BlockSpec(ctab.shape, fixed),
        pl.BlockSpec(w1.shape, fixed),
        pl.BlockSpec(b1.shape, fixed),
        pl.BlockSpec(w2.shape, fixed),
        pl.BlockSpec(wgcb.shape, fixed),
        pl.BlockSpec(wgfb.shape, fixed),
        pl.BlockSpec(jm.shape, fixed),
        pl.BlockSpec(bg2.shape, fixed),
        pl.BlockSpec(gamma2.shape, fixed),
        pl.BlockSpec(beta2.shape, fixed),
    ]
    out_shape = jax.ShapeDtypeStruct((B, D), jnp.float32)
    out_specs = pl.BlockSpec((_BLK, D), row_blk)
    return pl.pallas_call(
        _dense_body,
        grid=(nb,),
        in_specs=in_specs,
        out_specs=out_specs,
        out_shape=out_shape,
    )(maskf, clsf, desc, chem_emb, ctab, w1, b1, w2, wgcb, wgfb, jm, bg2,
      gamma2, beta2)


def kernel(chemical_idx, class_idx, descriptors, chem_table, class_table,
           W1, b1, W2, b2, Wg, bg, gamma, beta):
    B = chemical_idx.shape[0]
    V, D = chem_table.shape

    idx32 = chemical_idx.astype(jnp.int32)
    chem_emb = _make_sc_gather(V, D, B)(chem_table, idx32)

    is_unknown = chemical_idx == 0
    maskf = 1.0 - is_unknown.astype(jnp.float32)
    clsf = class_idx.astype(jnp.float32)
    nclass = class_table.shape[0]
    pad = (-nclass) % 32
    ctab = jnp.pad(class_table + b2.reshape(1, D), ((0, pad), (0, 0)))
    ones_row = jnp.ones((1, D), jnp.float32)
    wgcb = Wg[:D].reshape(D, 1) @ ones_row
    wgfb = Wg[D:].reshape(D, 1) @ ones_row
    jm = jnp.full((D, D), 1.0 / D, jnp.float32)
    out = _dense_forward(
        maskf, clsf, descriptors, chem_emb, ctab, W1, b1.reshape(1, -1), W2,
        wgcb, wgfb, jm, bg.reshape(1, 1), gamma.reshape(1, D),
        beta.reshape(1, D))
    return out, is_unknown


# speedup vs baseline: 1.0488x; 1.0155x over previous
import functools

import jax
import jax.numpy as jnp
from jax import lax
from jax.experimental import pallas as pl
from jax.experimental.pallas import tpu as pltpu
from jax.experimental.pallas import tpu_sc as plsc

_BLK = 4096
_IDX_CHUNK = 128


@functools.lru_cache(maxsize=None)
def _make_sc_gather(V, D, B):
    info = plsc.get_sparse_core_info()
    NC, NS = info.num_cores, info.num_subcores
    NW = NC * NS
    bw = B // NW
    nch = bw // _IDX_CHUNK
    mesh = plsc.VectorSubcoreMesh(core_axis_name="c", subcore_axis_name="s")

    @functools.partial(
        pl.kernel,
        mesh=mesh,
        out_type=jax.ShapeDtypeStruct((B, D), jnp.float32),
        scratch_types=[
            pltpu.VMEM((bw,), jnp.int32),
            pltpu.VMEM((bw, D), jnp.float32),
            pltpu.SemaphoreType.DMA((8,)),
            pltpu.SemaphoreType.DMA,
        ],
    )
    def gather_k(table_hbm, idx_hbm, out_hbm, idx_v, rows_v, gsem, wsem):
        wid = lax.axis_index("s") * NC + lax.axis_index("c")
        base = wid * bw
        pltpu.sync_copy(idx_hbm.at[pl.ds(base, bw)], idx_v)
        gathers = [
            pltpu.async_copy(
                table_hbm.at[idx_v.at[pl.ds(j * _IDX_CHUNK, _IDX_CHUNK)]],
                rows_v.at[pl.ds(j * _IDX_CHUNK, _IDX_CHUNK)],
                gsem.at[j],
            )
            for j in range(nch)
        ]
        writes = []
        for j in range(nch):
            gathers[j].wait()
            writes.append(pltpu.async_copy(
                rows_v.at[pl.ds(j * _IDX_CHUNK, _IDX_CHUNK)],
                out_hbm.at[pl.ds(base + j * _IDX_CHUNK, _IDX_CHUNK)],
                wsem,
            ))
        for w in writes:
            w.wait()

    return gather_k


def _dense_body(idx_ref, cls_ref, desc_ref, chem_ref, ctab_ref, w1_ref,
                b1_ref, w2_ref, b2_ref, wg_ref, bg_ref,
                gamma_ref, beta_ref, out_ref):
    blk, d = chem_ref.shape
    chem = chem_ref[...]
    desc = desc_ref[...]

    maskf = (idx_ref[...] != 0).astype(jnp.float32)
    clsf = cls_ref[...].astype(jnp.float32)
    mask_col = maskf.reshape(blk, 1)
    cls_col = clsf.reshape(blk, 1)

    nclass = ctab_ref.shape[0]
    ones_c = jnp.ones((1, nclass), jnp.float32)
    cls_dense = jnp.dot(cls_col, ones_c, preferred_element_type=jnp.float32)
    onehot = (cls_dense == lax.broadcasted_iota(
        jnp.int32, cls_dense.shape, 1).astype(jnp.float32)).astype(jnp.float32)
    ctab = ctab_ref[...] + b2_ref[...].reshape(1, d)
    cls_emb = jnp.dot(onehot, ctab, preferred_element_type=jnp.float32)

    h = jnp.maximum(
        jnp.dot(desc, w1_ref[...], preferred_element_type=jnp.float32)
        + b1_ref[...].reshape(1, -1), 0.0)
    fallback = (jnp.dot(h, w2_ref[...], preferred_element_type=jnp.float32)
                + cls_emb)

    ones_d = jnp.ones((1, d), jnp.float32)
    wgcb = jnp.dot(wg_ref[0:d, :], ones_d, preferred_element_type=jnp.float32)
    wgfb = jnp.dot(wg_ref[d:2 * d, :], ones_d, preferred_element_type=jnp.float32)
    logit = (jnp.dot(chem, wgcb, preferred_element_type=jnp.float32)
             + jnp.dot(fallback, wgfb, preferred_element_type=jnp.float32)
             + bg_ref[0])
    mask_dense = jnp.dot(mask_col, ones_d, preferred_element_type=jnp.float32)
    alpha = mask_dense / (1.0 + jnp.exp(-logit))

    emb = fallback + alpha * (chem - fallback)
    jm = jnp.full((d, d), 1.0 / d, jnp.float32)
    mean = jnp.dot(emb, jm, preferred_element_type=jnp.float32)
    cent = emb - mean
    var = jnp.dot(cent * cent, jm, preferred_element_type=jnp.float32)
    out_ref[...] = (cent * lax.rsqrt(var + 1e-5) * gamma_ref[...].reshape(1, d)
                    + beta_ref[...].reshape(1, d))


def _dense_forward(idx32, cls32, desc, chem_emb, ctab, w1, b1, w2, b2,
                   wg, bg, gamma, beta):
    B, D = chem_emb.shape
    nd = desc.shape[1]
    nb = B // _BLK

    def row_blk(i):
        return (i, 0)

    def row_blk1(i):
        return (i,)

    def fixed(i):
        return (0, 0)

    def fixed1(i):
        return (0,)

    in_specs = [
        pl.BlockSpec((_BLK,), row_blk1),
        pl.BlockSpec((_BLK,), row_blk1),
        pl.BlockSpec((_BLK, nd), row_blk),
        pl.BlockSpec((_BLK, D), row_blk),
        pl.BlockSpec(ctab.shape, fixed),
        pl.BlockSpec(w1.shape, fixed),
        pl.BlockSpec(b1.shape, fixed1),
        pl.BlockSpec(w2.shape, fixed),
        pl.BlockSpec(b2.shape, fixed1),
        pl.BlockSpec(wg.shape, fixed),
        pl.BlockSpec(bg.shape, fixed1),
        pl.BlockSpec(gamma.shape, fixed1),
        pl.BlockSpec(beta.shape, fixed1),
    ]
    out_shape = jax.ShapeDtypeStruct((B, D), jnp.float32)
    out_specs = pl.BlockSpec((_BLK, D), row_blk)
    return pl.pallas_call(
        _dense_body,
        grid=(nb,),
        in_specs=in_specs,
        out_specs=out_specs,
        out_shape=out_shape,
    )(idx32, cls32, desc, chem_emb, ctab, w1, b1, w2, b2, wg, bg, gamma, beta)


def kernel(chemical_idx, class_idx, descriptors, chem_table, class_table,
           W1, b1, W2, b2, Wg, bg, gamma, beta):
    B = chemical_idx.shape[0]
    V, D = chem_table.shape

    idx32 = chemical_idx.astype(jnp.int32)
    cls32 = class_idx.astype(jnp.int32)
    chem_emb = _make_sc_gather(V, D, B)(chem_table, idx32)

    out = _dense_forward(idx32, cls32, descriptors, chem_emb, class_table,
                         W1, b1, W2, b2, Wg, bg, gamma, beta)
    return out, chemical_idx == 0
